# Initial kernel scaffold; baseline (speedup 1.0000x reference)
import jax
import jax.numpy as jnp
from jax.experimental import pallas as pl


def kernel(x, weight):
    raise NotImplementedError("write your pallas kernel here")


# speedup vs baseline: 4.7433x; 4.7433x over previous
import functools

import jax
import jax.numpy as jnp
from jax import lax
from jax.experimental import pallas as pl
from jax.experimental.pallas import tpu as pltpu
from jax.experimental.pallas import tpu_sc as plsc

NUM_EMB = 100000
D = 64
B_ROWS = 16384
B_COLS = 200
B_TOT = B_ROWS * B_COLS
IDX_MINOR = 128
IDX_MAJOR = B_TOT // IDX_MINOR

NW = 32
ROWS_PER_W = IDX_MAJOR // NW
ROWS_PER_CHUNK = 4
CHUNK = ROWS_PER_CHUNK * IDX_MINOR
N_CHUNKS = ROWS_PER_W // ROWS_PER_CHUNK

_mesh = plsc.VectorSubcoreMesh(core_axis_name="c", subcore_axis_name="s")


@functools.partial(
    pl.kernel,
    mesh=_mesh,
    compiler_params=pltpu.CompilerParams(use_tc_tiling_on_sc=False),
    out_type=jax.ShapeDtypeStruct((B_TOT, D), jnp.float32),
    scratch_types=[
        pltpu.VMEM((ROWS_PER_CHUNK, IDX_MINOR), jnp.int32),
        pltpu.VMEM((CHUNK, D), jnp.float32),
        pltpu.SemaphoreType.DMA,
    ],
)
def _emb_lookup(idx_hbm, table_hbm, out_hbm, idx_v, rows_v, sem):
    wid = lax.axis_index("s") * 2 + lax.axis_index("c")
    w_row0 = wid * ROWS_PER_W

    def body(i, carry):
        row0 = w_row0 + i * ROWS_PER_CHUNK
        pltpu.sync_copy(idx_hbm.at[pl.ds(row0, ROWS_PER_CHUNK)], idx_v)
        copies = [
            pltpu.async_copy(
                table_hbm.at[idx_v.at[j]],
                rows_v.at[pl.ds(j * IDX_MINOR, IDX_MINOR)],
                sem,
            )
            for j in range(ROWS_PER_CHUNK)
        ]
        for c in copies:
            c.wait()
        pltpu.sync_copy(rows_v, out_hbm.at[pl.ds(row0 * IDX_MINOR, CHUNK)])
        return carry

    lax.fori_loop(0, N_CHUNKS, body, 0)


def kernel(x, weight):
    idx = x.reshape(IDX_MAJOR, IDX_MINOR).astype(jnp.int32)
    out = _emb_lookup(idx, weight)
    return out.reshape(B_ROWS, B_COLS, D)


# speedup vs baseline: 5.1636x; 1.0886x over previous
import functools

import jax
import jax.numpy as jnp
from jax import lax
from jax.experimental import pallas as pl
from jax.experimental.pallas import tpu as pltpu
from jax.experimental.pallas import tpu_sc as plsc

NUM_EMB = 100000
D = 64
B_ROWS = 16384
B_COLS = 200
B_TOT = B_ROWS * B_COLS

NW = 32
B_PER_W = B_TOT // NW
IDXW = 128
GPC = 4
CHUNK = GPC * IDXW
N_PAIR = B_PER_W // (2 * CHUNK)
IDX_ROWS_W = B_PER_W // IDXW

_mesh = plsc.VectorSubcoreMesh(core_axis_name="c", subcore_axis_name="s")


@functools.partial(
    pl.kernel,
    mesh=_mesh,
    compiler_params=pltpu.CompilerParams(use_tc_tiling_on_sc=False),
    out_type=jax.ShapeDtypeStruct((B_TOT, D), jnp.float32),
    scratch_types=[
        pltpu.VMEM((GPC, IDXW), jnp.int32),
        pltpu.VMEM((GPC, IDXW), jnp.int32),
        pltpu.VMEM((CHUNK, D), jnp.float32),
        pltpu.VMEM((CHUNK, D), jnp.float32),
        pltpu.SemaphoreType.DMA,
        pltpu.SemaphoreType.DMA,
        pltpu.SemaphoreType.DMA,
        pltpu.SemaphoreType.DMA,
    ],
)
def _emb_lookup(idx_hbm, table_hbm, out_hbm, ib0, ib1, rows0, rows1,
                sem_g0, sem_g1, sem_s0, sem_s1):
    wid = lax.axis_index("s") * 2 + lax.axis_index("c")
    w_row0 = wid * IDX_ROWS_W
    w0 = wid * B_PER_W

    def idx_load(c, ib):
        pltpu.sync_copy(idx_hbm.at[pl.ds(w_row0 + c * GPC, GPC)], ib)

    def gathers(ib, rows, sem):
        for j in range(GPC):
            pltpu.make_async_copy(
                table_hbm.at[ib.at[j]],
                rows.at[pl.ds(j * IDXW, IDXW)],
                sem,
            ).start()

    def gathers_wait(ib, rows, sem):
        for j in range(GPC):
            pltpu.make_async_copy(
                table_hbm.at[ib.at[j]],
                rows.at[pl.ds(j * IDXW, IDXW)],
                sem,
            ).wait()

    def store(c, rows, sem):
        return pltpu.make_async_copy(
            rows,
            out_hbm.at[pl.ds(w0 + c * CHUNK, CHUNK)],
            sem,
        )

    def pair(k, first):
        c0 = 2 * k
        c1 = c0 + 1
        idx_load(c0, ib0)
        if not first:
            store(0, rows0, sem_s0).wait()
        gathers(ib0, rows0, sem_g0)
        idx_load(c1, ib1)
        if not first:
            store(0, rows1, sem_s1).wait()
        gathers(ib1, rows1, sem_g1)
        gathers_wait(ib0, rows0, sem_g0)
        store(c0, rows0, sem_s0).start()
        gathers_wait(ib1, rows1, sem_g1)
        store(c1, rows1, sem_s1).start()

    pair(0, True)

    def body(k, carry):
        pair(k, False)
        return carry

    lax.fori_loop(1, N_PAIR, body, 0)

    store(0, rows0, sem_s0).wait()
    store(0, rows1, sem_s1).wait()


def kernel(x, weight):
    idx = x.reshape(B_TOT // IDXW, IDXW).astype(jnp.int32)
    out = _emb_lookup(idx, weight)
    return out.reshape(B_ROWS, B_COLS, D)


# speedup vs baseline: 6.9769x; 1.3512x over previous
import functools

import jax
import jax.numpy as jnp
from jax import lax
from jax.experimental import pallas as pl
from jax.experimental.pallas import tpu as pltpu
from jax.experimental.pallas import tpu_sc as plsc

NUM_EMB = 100000
D = 64
B_ROWS = 16384
B_COLS = 200

NW = 32
BPW = B_ROWS // NW
NBB = BPW // 128
NTG = 8
NG = NBB * (B_COLS // NTG)

_mesh = plsc.VectorSubcoreMesh(core_axis_name="c", subcore_axis_name="s")


@functools.partial(
    pl.kernel,
    mesh=_mesh,
    compiler_params=pltpu.CompilerParams(use_tc_tiling_on_sc=False, needs_layout_passes=False),
    out_type=jax.ShapeDtypeStruct((B_COLS, 8, 128, 1024), jnp.float32),
    scratch_types=[
        pltpu.VMEM((NTG, 128), jnp.int32),
        pltpu.VMEM((128, D), jnp.float32),
        pltpu.VMEM((128, D), jnp.float32),
        pltpu.VMEM((8, 1024), jnp.float32),
        pltpu.VMEM((8, 1024), jnp.float32),
        pltpu.SemaphoreType.DMA,
        pltpu.SemaphoreType.DMA,
        pltpu.SemaphoreType.DMA,
        pltpu.SemaphoreType.DMA,
    ],
)
def _emb_lookup(xt_hbm, table_hbm, out_hbm, xblk, g0, g1, s0, s1,
                sem_g0, sem_g1, sem_s0, sem_s1):
    wid = lax.axis_index("s") * 2 + lax.axis_index("c")
    w_b0 = wid * BPW
    w_bt0 = wid * NBB
    lane = lax.iota(jnp.int32, 16)
    rows = [lane + (grp * 16) for grp in range(8)]

    def gather(tt, gb, sem):
        return pltpu.make_async_copy(
            table_hbm.at[xblk.at[tt]], gb, sem)

    def store(t, bt, sb, sem):
        return pltpu.make_async_copy(
            sb, out_hbm.at[t, pl.ds(0, 8), bt], sem)

    def transpose(gb, sb):
        def dbody(d, carry):
            col = (jnp.full((16,), d, jnp.int32) + lane) & 63
            flatbase = col << 7
            for grp in range(8):
                flat = flatbase + rows[grp]
                i0 = flat >> 10
                i1 = flat & 1023
                v = plsc.load_gather(gb, [rows[grp], col])
                plsc.store_scatter(sb, [i0, i1], v)
            return carry

        lax.fori_loop(0, D, dbody, 0)

    def group(gidx, first):
        bb = gidx // (B_COLS // NTG)
        tg = gidx - bb * (B_COLS // NTG)
        t0 = tg * NTG
        b0 = w_b0 + bb * 128
        bt = w_bt0 + bb
        pltpu.sync_copy(
            xt_hbm.at[pl.ds(t0, NTG), pl.ds(b0, 128)], xblk)
        gather(0, g0, sem_g0).start()
        for tt in range(NTG):
            gb, sg = (g0, sem_g0) if tt % 2 == 0 else (g1, sem_g1)
            sb, ss = (s0, sem_s0) if tt % 2 == 0 else (s1, sem_s1)
            nb, sn = (g1, sem_g1) if tt % 2 == 0 else (g0, sem_g0)
            if tt + 1 < NTG:
                gather(tt + 1, nb, sn).start()
            gather(tt, gb, sg).wait()
            if not (first and tt < 2):
                store(0, 0, sb, ss).wait()
            transpose(gb, sb)
            store(t0 + tt, bt, sb, ss).start()

    group(0, True)

    def body(gidx, carry):
        group(gidx, False)
        return carry

    lax.fori_loop(1, NG, body, 0)

    store(0, 0, s0, sem_s0).wait()
    store(0, 0, s1, sem_s1).wait()


def kernel(x, weight):
    xt = jnp.transpose(x).astype(jnp.int32)
    a = _emb_lookup(xt, weight)
    a5 = a.reshape(B_COLS, 8, 128, 8, 128)
    return jnp.transpose(a5, (2, 4, 0, 1, 3)).reshape(B_ROWS, B_COLS, D)
